# Initial kernel scaffold; baseline (speedup 1.0000x reference)
import jax
import jax.numpy as jnp
from jax.experimental import pallas as pl


def kernel(x, edge_index, edge_attr, batch, params):
    raise NotImplementedError("write your pallas kernel here")


# speedup vs baseline: 2.4814x; 2.4814x over previous
import functools

import jax
import jax.numpy as jnp
from jax import lax
from jax.experimental import pallas as pl
from jax.experimental.pallas import tpu as pltpu
from jax.experimental.pallas import tpu_sc as plsc

F32 = jnp.float32


def _matmul_bias_body(x_ref, w_ref, b_ref, o_ref):
    o_ref[...] = (
        jnp.dot(x_ref[...], w_ref[...], preferred_element_type=F32) + b_ref[...]
    )


def _matmul_bias(x, w, b, blk):
    n, k = x.shape
    _, m = w.shape
    grid = n // blk
    return pl.pallas_call(
        _matmul_bias_body,
        grid=(grid,),
        in_specs=[
            pl.BlockSpec((blk, k), lambda i: (i, 0)),
            pl.BlockSpec((k, m), lambda i: (0, 0)),
            pl.BlockSpec((1, m), lambda i: (0, 0)),
        ],
        out_specs=pl.BlockSpec((blk, m), lambda i: (i, 0)),
        out_shape=jax.ShapeDtypeStruct((n, m), F32),
    )(x, w, b.reshape(1, m))


def _mlp_body(h_ref, a0_ref, a1_ref, w1_ref, b1_ref, w2_ref, b2_ref, o_ref,
              *, final):
    h_in = h_ref[...] + a0_ref[...] + a1_ref[...]
    mid = jnp.maximum(
        jnp.dot(h_in, w1_ref[...], preferred_element_type=F32) + b1_ref[...], 0.0
    )
    out = jnp.dot(mid, w2_ref[...], preferred_element_type=F32) + b2_ref[...]
    if not final:
        out = jnp.maximum(out, 0.0)
    o_ref[...] = out


def _gin_mlp(h, agg2, w1, b1, w2, b2, final, blk):
    n, d = h.shape
    dh = w1.shape[1]
    grid = n // blk
    nblk = n // blk
    return pl.pallas_call(
        functools.partial(_mlp_body, final=final),
        grid=(grid,),
        in_specs=[
            pl.BlockSpec((blk, d), lambda i: (i, 0)),
            pl.BlockSpec((blk, d), lambda i: (i, 0)),
            pl.BlockSpec((blk, d), lambda i, _n=nblk: (i + _n, 0)),
            pl.BlockSpec((d, dh), lambda i: (0, 0)),
            pl.BlockSpec((1, dh), lambda i: (0, 0)),
            pl.BlockSpec((dh, d), lambda i: (0, 0)),
            pl.BlockSpec((1, d), lambda i: (0, 0)),
        ],
        out_specs=pl.BlockSpec((blk, d), lambda i: (i, 0)),
        out_shape=jax.ShapeDtypeStruct((n, d), F32),
    )(h, agg2, agg2, w1, b1.reshape(1, dh), w2, b2.reshape(1, d))


def _mish_tc(x):
    sp = jnp.maximum(x, 0.0) + jnp.log1p(jnp.exp(-jnp.abs(x)))
    return x * jnp.tanh(sp)


def _pool_body(node_ref, batch_ref, gpw_ref, gpb_ref, nfwb_ref,
               grep_ref, gout_ref, gtnp_ref, acc_ref, cnt_ref, *, ngraph, nsteps):
    i = pl.program_id(0)

    @pl.when(i == 0)
    def _():
        acc_ref[...] = jnp.zeros_like(acc_ref)
        cnt_ref[...] = jnp.zeros_like(cnt_ref)

    bblk = batch_ref[0, 0, :]
    onehot = (
        lax.broadcasted_iota(jnp.int32, (ngraph, bblk.shape[0]), 0)
        == bblk[None, :]
    ).astype(F32)
    acc_ref[...] += jnp.dot(onehot, node_ref[...], preferred_element_type=F32)
    cnt_ref[...] += jnp.sum(onehot, axis=1, keepdims=True)

    @pl.when(i == nsteps - 1)
    def _():
        grep = acc_ref[...] / jnp.maximum(cnt_ref[...], 1.0)
        grep_ref[...] = grep
        gout_ref[...] = (
            jnp.dot(grep, gpw_ref[...], preferred_element_type=F32) + gpb_ref[...]
        )
        gtnp_ref[...] = jnp.dot(grep, nfwb_ref[...], preferred_element_type=F32)


def _pool(node_rep, batch3, gp_w, gp_b, nf_w_bot, blk):
    n, d = node_rep.shape
    g = gp_w.shape[0] if gp_w.shape[0] != d else 256
    g = 256
    nsteps = n // blk
    return pl.pallas_call(
        functools.partial(_pool_body, ngraph=g, nsteps=nsteps),
        grid=(nsteps,),
        in_specs=[
            pl.BlockSpec((blk, d), lambda i: (i, 0)),
            pl.BlockSpec((1, 1, blk), lambda i: (i, 0, 0)),
            pl.BlockSpec((d, d), lambda i: (0, 0)),
            pl.BlockSpec((1, d), lambda i: (0, 0)),
            pl.BlockSpec((d, d), lambda i: (0, 0)),
        ],
        out_specs=[
            pl.BlockSpec((g, d), lambda i: (0, 0)),
            pl.BlockSpec((g, d), lambda i: (0, 0)),
            pl.BlockSpec((g, d), lambda i: (0, 0)),
        ],
        out_shape=[
            jax.ShapeDtypeStruct((g, d), F32),
            jax.ShapeDtypeStruct((g, d), F32),
            jax.ShapeDtypeStruct((g, d), F32),
        ],
        scratch_shapes=[
            pltpu.VMEM((g, d), F32),
            pltpu.VMEM((g, 1), F32),
        ],
    )(node_rep, batch3, gp_w, gp_b.reshape(1, d), nf_w_bot)


def _heads_body(node_ref, batch_ref, gtnp_ref, nfwt_ref, nfb_ref,
                efwt_ref, efwb_ref, efb_ref,
                n2_ref, se_ref, a_ref, bt_ref, *, ngraph):
    bblk = batch_ref[0, 0, :]
    onehot = (
        lax.broadcasted_iota(jnp.int32, (bblk.shape[0], ngraph), 1)
        == bblk[:, None]
    ).astype(F32)
    gtn = jnp.dot(onehot, gtnp_ref[...], preferred_element_type=F32)
    pre = (
        jnp.dot(node_ref[...], nfwt_ref[...], preferred_element_type=F32)
        + gtn + nfb_ref[...]
    )
    n2 = _mish_tc(pre)
    n2_ref[...] = n2
    a = jnp.dot(n2, efwt_ref[...], preferred_element_type=F32)
    bt = jnp.dot(n2, efwb_ref[...], preferred_element_type=F32) + efb_ref[...]
    a_ref[...] = a
    bt_ref[...] = bt
    se_ref[...] = _mish_tc(a + bt)


def _heads(node_rep, batch3, gtnp, nf_w_top, nf_b, ef_w_top, ef_w_bot, ef_b, blk):
    n, d = node_rep.shape
    g = gtnp.shape[0]
    nsteps = n // blk
    outs = pl.pallas_call(
        functools.partial(_heads_body, ngraph=g),
        grid=(nsteps,),
        in_specs=[
            pl.BlockSpec((blk, d), lambda i: (i, 0)),
            pl.BlockSpec((1, 1, blk), lambda i: (i, 0, 0)),
            pl.BlockSpec((g, d), lambda i: (0, 0)),
            pl.BlockSpec((d, d), lambda i: (0, 0)),
            pl.BlockSpec((1, d), lambda i: (0, 0)),
            pl.BlockSpec((d, d), lambda i: (0, 0)),
            pl.BlockSpec((d, d), lambda i: (0, 0)),
            pl.BlockSpec((1, d), lambda i: (0, 0)),
        ],
        out_specs=[pl.BlockSpec((blk, d), lambda i: (i, 0))] * 4,
        out_shape=[jax.ShapeDtypeStruct((n, d), F32)] * 4,
    )(node_rep, batch3, gtnp, nf_w_top, nf_b.reshape(1, d),
      ef_w_top, ef_w_bot, ef_b.reshape(1, d))
    return outs


_EBLK = 128


def _msgpass_body(h_hbm, e_hbm, src_hbm, dst_hbm, zeros_hbm, out_hbm,
                  agg, src_v, dst_v, rows_v, e_v, sem,
                  *, n_nodes, n_pad, n_edges, emb):
    c = lax.axis_index("c")
    s = lax.axis_index("s")
    half = n_edges // 2
    nblk = half // _EBLK
    rows_per_tile = n_pad // 16

    zlo = s * rows_per_tile
    pltpu.sync_copy(zeros_hbm.at[pl.ds(zlo, rows_per_tile)],
                    agg.at[pl.ds(zlo, rows_per_tile)])
    plsc.subcore_barrier()

    base_e = c * half
    ntile = (nblk - s + 15) // 16

    def blk_body(i, _):
        b = s + i * 16
        e0 = base_e + b * _EBLK
        pltpu.sync_copy(src_hbm.at[pl.ds(e0, _EBLK)], src_v)
        pltpu.sync_copy(dst_hbm.at[pl.ds(e0, _EBLK)], dst_v)
        cp = pltpu.async_copy(h_hbm.at[src_v], rows_v, sem)
        pltpu.sync_copy(e_hbm.at[pl.ds(e0, _EBLK)], e_v)
        cp.wait()

        def row_body(r, _):
            for j in range(emb // 16):
                sl = pl.ds(j * 16, 16)
                e_v[r, sl] = jnp.maximum(rows_v[r, sl] + e_v[r, sl], 0.0)
            return 0

        lax.fori_loop(0, _EBLK, row_body, 0)
        pltpu.sync_copy(e_v, agg.at[dst_v], add=True)
        return 0

    lax.fori_loop(0, ntile, blk_body, 0)
    plsc.subcore_barrier()
    chunk = 80
    for k in range(rows_per_tile // chunk):
        off = zlo + k * chunk

        @pl.when(off < n_nodes)
        def _():
            pltpu.sync_copy(agg.at[pl.ds(off, chunk)],
                            out_hbm.at[pl.ds(c * n_nodes + off, chunk)])


def _msgpass(h, e, src, dst, zeros):
    n, emb = h.shape
    n_pad = zeros.shape[0]
    n_edges = e.shape[0]
    mesh = plsc.VectorSubcoreMesh(core_axis_name="c", subcore_axis_name="s")
    body = functools.partial(_msgpass_body, n_nodes=n, n_pad=n_pad,
                             n_edges=n_edges, emb=emb)
    return pl.kernel(
        body,
        out_type=jax.ShapeDtypeStruct((2 * n, emb), F32),
        mesh=mesh,
        scratch_types=[
            pltpu.VMEM_SHARED((n_pad, emb), F32),
            pltpu.VMEM((_EBLK,), jnp.int32),
            pltpu.VMEM((_EBLK,), jnp.int32),
            pltpu.VMEM((_EBLK, emb), F32),
            pltpu.VMEM((_EBLK, emb), F32),
            pltpu.SemaphoreType.DMA,
        ],
    )(h, e, src, dst, zeros)


def _mish_sc(x):
    ex = jnp.exp(jnp.minimum(x, 40.0))
    t = ex * (ex + 2.0)
    return x * (t / (t + 2.0))


def _edgemix_body(a_hbm, b_hbm, src_hbm, dst_hbm, out_hbm,
                  src_v, dst_v, arow_v, brow_v, out_v, sem_a, sem_b,
                  *, n_edges, emb):
    c = lax.axis_index("c")
    s = lax.axis_index("s")
    w = s * 2 + c
    nblk = n_edges // _EBLK
    ntile = (nblk - w + 31) // 32

    def blk_body(i, _):
        b = w + i * 32
        e0 = b * _EBLK
        pltpu.sync_copy(src_hbm.at[pl.ds(e0, _EBLK)], src_v)
        pltpu.sync_copy(dst_hbm.at[pl.ds(e0, _EBLK)], dst_v)
        ca = pltpu.async_copy(a_hbm.at[src_v], arow_v, sem_a)
        cb = pltpu.async_copy(b_hbm.at[dst_v], brow_v, sem_b)
        ca.wait()
        cb.wait()

        def row_body(r, _):
            for j in range(emb // 16):
                sl = pl.ds(j * 16, 16)
                x0 = arow_v[2 * r, sl] + brow_v[2 * r, sl]
                x1 = arow_v[2 * r + 1, sl] + brow_v[2 * r + 1, sl]
                out_v[r, sl] = 0.5 * (_mish_sc(x0) + _mish_sc(x1))
            return 0

        lax.fori_loop(0, _EBLK // 2, row_body, 0)
        pltpu.sync_copy(out_v, out_hbm.at[pl.ds(b * (_EBLK // 2), _EBLK // 2)])
        return 0

    lax.fori_loop(0, ntile, blk_body, 0)


def _edgemix(a, bt, src, dst):
    n, emb = a.shape
    n_edges = src.shape[0]
    mesh = plsc.VectorSubcoreMesh(core_axis_name="c", subcore_axis_name="s")
    body = functools.partial(_edgemix_body, n_edges=n_edges, emb=emb)
    return pl.kernel(
        body,
        out_type=jax.ShapeDtypeStruct((n_edges // 2, emb), F32),
        mesh=mesh,
        scratch_types=[
            pltpu.VMEM((_EBLK,), jnp.int32),
            pltpu.VMEM((_EBLK,), jnp.int32),
            pltpu.VMEM((_EBLK, emb), F32),
            pltpu.VMEM((_EBLK, emb), F32),
            pltpu.VMEM((_EBLK // 2, emb), F32),
            pltpu.SemaphoreType.DMA,
            pltpu.SemaphoreType.DMA,
        ],
    )(a, bt, src, dst)


def kernel(x, edge_index, edge_attr, batch, params):
    n, _ = x.shape
    emb = params['atom_w'].shape[1]
    num_layers = params['edge_w'].shape[0]
    src = edge_index[0]
    dst = edge_index[1]
    nblk = 1000
    batch3 = batch.reshape(n // nblk, 1, nblk)
    n_pad = ((n + 16 * 80 - 1) // (16 * 80)) * (16 * 80)
    zeros = jnp.zeros((n_pad, emb), F32)

    h = _matmul_bias(x, params['atom_w'], params['atom_b'], nblk)
    for l in range(num_layers):
        e = _matmul_bias(edge_attr, params['edge_w'][l], params['edge_b'][l], 4000)
        agg2 = _msgpass(h, e, src, dst, zeros)
        h = _gin_mlp(h, agg2, params['mlp_w1'][l], params['mlp_b1'][l],
                     params['mlp_w2'][l], params['mlp_b2'][l],
                     final=(l == num_layers - 1), blk=nblk)

    nf_w = params['nf_w']
    ef_w = params['ef_w']
    _, graph_out, gtnp = _pool(h, batch3, params['gp_w'], params['gp_b'],
                               nf_w[emb:], nblk)
    n2, se, a, bt = _heads(h, batch3, gtnp, nf_w[:emb], params['nf_b'],
                           ef_w[:emb], ef_w[emb:], params['ef_b'], nblk)
    edge_rep = _edgemix(a, bt, src, dst)
    return (n2, se, edge_rep, graph_out)


# speedup vs baseline: 3.1571x; 1.2723x over previous
import functools

import jax
import jax.numpy as jnp
from jax import lax
from jax.experimental import pallas as pl
from jax.experimental.pallas import tpu as pltpu
from jax.experimental.pallas import tpu_sc as plsc

F32 = jnp.float32


def _matmul_bias_body(x_ref, w_ref, b_ref, o_ref):
    o_ref[...] = (
        jnp.dot(x_ref[...], w_ref[...], preferred_element_type=F32) + b_ref[...]
    )


def _matmul_bias(x, w, b, blk):
    n, k = x.shape
    _, m = w.shape
    grid = n // blk
    return pl.pallas_call(
        _matmul_bias_body,
        grid=(grid,),
        in_specs=[
            pl.BlockSpec((blk, k), lambda i: (i, 0)),
            pl.BlockSpec((k, m), lambda i: (0, 0)),
            pl.BlockSpec((1, m), lambda i: (0, 0)),
        ],
        out_specs=pl.BlockSpec((blk, m), lambda i: (i, 0)),
        out_shape=jax.ShapeDtypeStruct((n, m), F32),
    )(x, w, b.reshape(1, m))


def _mlp_body(h_ref, a0_ref, a1_ref, w1_ref, b1_ref, w2_ref, b2_ref, o_ref,
              *, final):
    h_in = h_ref[...] + a0_ref[...] + a1_ref[...]
    mid = jnp.maximum(
        jnp.dot(h_in, w1_ref[...], preferred_element_type=F32) + b1_ref[...], 0.0
    )
    out = jnp.dot(mid, w2_ref[...], preferred_element_type=F32) + b2_ref[...]
    if not final:
        out = jnp.maximum(out, 0.0)
    o_ref[...] = out


def _gin_mlp(h, agg2, w1, b1, w2, b2, final, blk):
    n, d = h.shape
    dh = w1.shape[1]
    grid = n // blk
    nblk = n // blk
    return pl.pallas_call(
        functools.partial(_mlp_body, final=final),
        grid=(grid,),
        in_specs=[
            pl.BlockSpec((blk, d), lambda i: (i, 0)),
            pl.BlockSpec((blk, d), lambda i: (i, 0)),
            pl.BlockSpec((blk, d), lambda i, _n=nblk: (i + _n, 0)),
            pl.BlockSpec((d, dh), lambda i: (0, 0)),
            pl.BlockSpec((1, dh), lambda i: (0, 0)),
            pl.BlockSpec((dh, d), lambda i: (0, 0)),
            pl.BlockSpec((1, d), lambda i: (0, 0)),
        ],
        out_specs=pl.BlockSpec((blk, d), lambda i: (i, 0)),
        out_shape=jax.ShapeDtypeStruct((n, d), F32),
    )(h, agg2, agg2, w1, b1.reshape(1, dh), w2, b2.reshape(1, d))


def _mish_tc(x):
    sp = jnp.maximum(x, 0.0) + jnp.log1p(jnp.exp(-jnp.abs(x)))
    return x * jnp.tanh(sp)


def _pool_body(node_ref, batch_ref, gpw_ref, gpb_ref, nfwb_ref,
               grep_ref, gout_ref, gtnp_ref, acc_ref, cnt_ref, *, ngraph, nsteps):
    i = pl.program_id(0)

    @pl.when(i == 0)
    def _():
        acc_ref[...] = jnp.zeros_like(acc_ref)
        cnt_ref[...] = jnp.zeros_like(cnt_ref)

    bblk = batch_ref[0, 0, :]
    onehot = (
        lax.broadcasted_iota(jnp.int32, (ngraph, bblk.shape[0]), 0)
        == bblk[None, :]
    ).astype(F32)
    acc_ref[...] += jnp.dot(onehot, node_ref[...], preferred_element_type=F32)
    cnt_ref[...] += jnp.sum(onehot, axis=1, keepdims=True)

    @pl.when(i == nsteps - 1)
    def _():
        grep = acc_ref[...] / jnp.maximum(cnt_ref[...], 1.0)
        grep_ref[...] = grep
        gout_ref[...] = (
            jnp.dot(grep, gpw_ref[...], preferred_element_type=F32) + gpb_ref[...]
        )
        gtnp_ref[...] = jnp.dot(grep, nfwb_ref[...], preferred_element_type=F32)


def _pool(node_rep, batch3, gp_w, gp_b, nf_w_bot, blk):
    n, d = node_rep.shape
    g = gp_w.shape[0] if gp_w.shape[0] != d else 256
    g = 256
    nsteps = n // blk
    return pl.pallas_call(
        functools.partial(_pool_body, ngraph=g, nsteps=nsteps),
        grid=(nsteps,),
        in_specs=[
            pl.BlockSpec((blk, d), lambda i: (i, 0)),
            pl.BlockSpec((1, 1, blk), lambda i: (i, 0, 0)),
            pl.BlockSpec((d, d), lambda i: (0, 0)),
            pl.BlockSpec((1, d), lambda i: (0, 0)),
            pl.BlockSpec((d, d), lambda i: (0, 0)),
        ],
        out_specs=[
            pl.BlockSpec((g, d), lambda i: (0, 0)),
            pl.BlockSpec((g, d), lambda i: (0, 0)),
            pl.BlockSpec((g, d), lambda i: (0, 0)),
        ],
        out_shape=[
            jax.ShapeDtypeStruct((g, d), F32),
            jax.ShapeDtypeStruct((g, d), F32),
            jax.ShapeDtypeStruct((g, d), F32),
        ],
        scratch_shapes=[
            pltpu.VMEM((g, d), F32),
            pltpu.VMEM((g, 1), F32),
        ],
    )(node_rep, batch3, gp_w, gp_b.reshape(1, d), nf_w_bot)


def _heads_body(node_ref, batch_ref, gtnp_ref, nfwt_ref, nfb_ref,
                efwt_ref, efwb_ref, efb_ref,
                n2_ref, se_ref, a_ref, bt_ref, *, ngraph):
    bblk = batch_ref[0, 0, :]
    onehot = (
        lax.broadcasted_iota(jnp.int32, (bblk.shape[0], ngraph), 1)
        == bblk[:, None]
    ).astype(F32)
    gtn = jnp.dot(onehot, gtnp_ref[...], preferred_element_type=F32)
    pre = (
        jnp.dot(node_ref[...], nfwt_ref[...], preferred_element_type=F32)
        + gtn + nfb_ref[...]
    )
    n2 = _mish_tc(pre)
    n2_ref[...] = n2
    a = jnp.dot(n2, efwt_ref[...], preferred_element_type=F32)
    bt = jnp.dot(n2, efwb_ref[...], preferred_element_type=F32) + efb_ref[...]
    a_ref[...] = a
    bt_ref[...] = bt
    se_ref[...] = _mish_tc(a + bt)


def _heads(node_rep, batch3, gtnp, nf_w_top, nf_b, ef_w_top, ef_w_bot, ef_b, blk):
    n, d = node_rep.shape
    g = gtnp.shape[0]
    nsteps = n // blk
    outs = pl.pallas_call(
        functools.partial(_heads_body, ngraph=g),
        grid=(nsteps,),
        in_specs=[
            pl.BlockSpec((blk, d), lambda i: (i, 0)),
            pl.BlockSpec((1, 1, blk), lambda i: (i, 0, 0)),
            pl.BlockSpec((g, d), lambda i: (0, 0)),
            pl.BlockSpec((d, d), lambda i: (0, 0)),
            pl.BlockSpec((1, d), lambda i: (0, 0)),
            pl.BlockSpec((d, d), lambda i: (0, 0)),
            pl.BlockSpec((d, d), lambda i: (0, 0)),
            pl.BlockSpec((1, d), lambda i: (0, 0)),
        ],
        out_specs=[pl.BlockSpec((blk, d), lambda i: (i, 0))] * 4,
        out_shape=[jax.ShapeDtypeStruct((n, d), F32)] * 4,
    )(node_rep, batch3, gtnp, nf_w_top, nf_b.reshape(1, d),
      ef_w_top, ef_w_bot, ef_b.reshape(1, d))
    return outs


_EBLK = 128


def _msgpass_body(h_hbm, e_hbm, src_hbm, dst_hbm, zeros_hbm, out_hbm,
                  agg, src_v, dst_v, rows_v, e_v, sem,
                  *, n_nodes, n_pad, n_edges, emb):
    c = lax.axis_index("c")
    s = lax.axis_index("s")
    half = n_edges // 2
    nblk = half // _EBLK
    rows_per_tile = n_pad // 16

    zlo = s * rows_per_tile
    pltpu.sync_copy(zeros_hbm.at[pl.ds(zlo, rows_per_tile)],
                    agg.at[pl.ds(zlo, rows_per_tile)])
    plsc.subcore_barrier()

    base_e = c * half
    ntile = (nblk - s + 15) // 16

    def blk_body(i, _):
        b = s + i * 16
        e0 = base_e + b * _EBLK
        pltpu.sync_copy(src_hbm.at[pl.ds(e0, _EBLK)], src_v)
        pltpu.sync_copy(dst_hbm.at[pl.ds(e0, _EBLK)], dst_v)
        cp = pltpu.async_copy(h_hbm.at[src_v], rows_v, sem)
        pltpu.sync_copy(e_hbm.at[pl.ds(e0, _EBLK)], e_v)
        cp.wait()

        def row_body(r, _):
            for j in range(emb // 16):
                sl = pl.ds(j * 16, 16)
                e_v[r, sl] = jnp.maximum(rows_v[r, sl] + e_v[r, sl], 0.0)
            return 0

        lax.fori_loop(0, _EBLK, row_body, 0)
        pltpu.sync_copy(e_v, agg.at[dst_v], add=True)
        return 0

    lax.fori_loop(0, ntile, blk_body, 0)
    plsc.subcore_barrier()
    chunk = 80
    for k in range(rows_per_tile // chunk):
        off = zlo + k * chunk

        @pl.when(off < n_nodes)
        def _():
            pltpu.sync_copy(agg.at[pl.ds(off, chunk)],
                            out_hbm.at[pl.ds(c * n_nodes + off, chunk)])


def _msgpass(h, e, src, dst, zeros):
    n, emb = h.shape
    n_pad = zeros.shape[0]
    n_edges = e.shape[0]
    mesh = plsc.VectorSubcoreMesh(core_axis_name="c", subcore_axis_name="s")
    body = functools.partial(_msgpass_body, n_nodes=n, n_pad=n_pad,
                             n_edges=n_edges, emb=emb)
    return pl.kernel(
        body,
        out_type=jax.ShapeDtypeStruct((2 * n, emb), F32),
        mesh=mesh,
        scratch_types=[
            pltpu.VMEM_SHARED((n_pad, emb), F32),
            pltpu.VMEM((_EBLK,), jnp.int32),
            pltpu.VMEM((_EBLK,), jnp.int32),
            pltpu.VMEM((_EBLK, emb), F32),
            pltpu.VMEM((_EBLK, emb), F32),
            pltpu.SemaphoreType.DMA,
        ],
    )(h, e, src, dst, zeros)


def _edgegather_body(a_hbm, b_hbm, src_hbm, dst_hbm, ag_hbm, bg_hbm,
                     src_v, dst_v, arow_v, brow_v, sem_a, sem_b,
                     *, n_edges, emb):
    c = lax.axis_index("c")
    s = lax.axis_index("s")
    w = s * 2 + c
    nblk = n_edges // _EBLK
    ntile = (nblk - w + 31) // 32

    def blk_body(i, _):
        b = w + i * 32
        e0 = b * _EBLK
        pltpu.sync_copy(src_hbm.at[pl.ds(e0, _EBLK)], src_v)
        pltpu.sync_copy(dst_hbm.at[pl.ds(e0, _EBLK)], dst_v)
        ca = pltpu.async_copy(a_hbm.at[src_v], arow_v, sem_a)
        cb = pltpu.async_copy(b_hbm.at[dst_v], brow_v, sem_b)
        ca.wait()
        cb.wait()
        pltpu.sync_copy(arow_v, ag_hbm.at[pl.ds(e0, _EBLK)])
        pltpu.sync_copy(brow_v, bg_hbm.at[pl.ds(e0, _EBLK)])
        return 0

    lax.fori_loop(0, ntile, blk_body, 0)


def _edgegather(a, bt, src, dst):
    n, emb = a.shape
    n_edges = src.shape[0]
    mesh = plsc.VectorSubcoreMesh(core_axis_name="c", subcore_axis_name="s")
    body = functools.partial(_edgegather_body, n_edges=n_edges, emb=emb)
    return pl.kernel(
        body,
        out_type=[jax.ShapeDtypeStruct((n_edges, emb), F32)] * 2,
        mesh=mesh,
        scratch_types=[
            pltpu.VMEM((_EBLK,), jnp.int32),
            pltpu.VMEM((_EBLK,), jnp.int32),
            pltpu.VMEM((_EBLK, emb), F32),
            pltpu.VMEM((_EBLK, emb), F32),
            pltpu.SemaphoreType.DMA,
            pltpu.SemaphoreType.DMA,
        ],
    )(a, bt, src, dst)


def _edgemish_body(ag_ref, bg_ref, o_ref):
    x = ag_ref[...] + bg_ref[...]
    m = _mish_tc(x)
    o_ref[...] = 0.5 * (m[:, 0, :] + m[:, 1, :])


def _edgemish(ag, bg, blk):
    half, emb = ag.shape[0] // 2, ag.shape[1]
    ag2 = ag.reshape(half, 2, emb)
    bg2 = bg.reshape(half, 2, emb)
    grid = half // blk
    return pl.pallas_call(
        _edgemish_body,
        grid=(grid,),
        in_specs=[
            pl.BlockSpec((blk, 2, emb), lambda i: (i, 0, 0)),
            pl.BlockSpec((blk, 2, emb), lambda i: (i, 0, 0)),
        ],
        out_specs=pl.BlockSpec((blk, emb), lambda i: (i, 0)),
        out_shape=jax.ShapeDtypeStruct((half, emb), F32),
    )(ag2, bg2)


def kernel(x, edge_index, edge_attr, batch, params):
    n, _ = x.shape
    emb = params['atom_w'].shape[1]
    num_layers = params['edge_w'].shape[0]
    src = edge_index[0]
    dst = edge_index[1]
    nblk = 1000
    batch3 = batch.reshape(n // nblk, 1, nblk)
    n_pad = ((n + 16 * 80 - 1) // (16 * 80)) * (16 * 80)
    zeros = jnp.zeros((n_pad, emb), F32)

    h = _matmul_bias(x, params['atom_w'], params['atom_b'], nblk)
    for l in range(num_layers):
        e = _matmul_bias(edge_attr, params['edge_w'][l], params['edge_b'][l], 4000)
        agg2 = _msgpass(h, e, src, dst, zeros)
        h = _gin_mlp(h, agg2, params['mlp_w1'][l], params['mlp_b1'][l],
                     params['mlp_w2'][l], params['mlp_b2'][l],
                     final=(l == num_layers - 1), blk=nblk)

    nf_w = params['nf_w']
    ef_w = params['ef_w']
    _, graph_out, gtnp = _pool(h, batch3, params['gp_w'], params['gp_b'],
                               nf_w[emb:], nblk)
    n2, se, a, bt = _heads(h, batch3, gtnp, nf_w[:emb], params['nf_b'],
                           ef_w[:emb], ef_w[emb:], params['ef_b'], nblk)
    ag, bg = _edgegather(a, bt, src, dst)
    edge_rep = _edgemish(ag, bg, 2000)
    return (n2, se, edge_rep, graph_out)
